# baseline (speedup 1.0000x reference)
import functools

import jax
import jax.numpy as jnp
from jax import lax
from jax.experimental import pallas as pl
from jax.experimental.pallas import tpu as pltpu
from jax.experimental.pallas import tpu_sc as plsc

N_NODES = 10000
N_EDGES = 320000
D = 128

NC = 2
NS = 16
NW = NC * NS
K = 80
NBUF = 2
NCHUNK0 = 100
NCHUNK1 = 152
EPW0 = NCHUNK0 * K
EPW1 = NCHUNK1 * K
EPW_MAX = EPW1
E_PAD = NS * (EPW0 + EPW1)
N_PAD = 10240
ROWS_PER_TILE = N_PAD // NS


def _sc_segment_sum(x, src1, dst1, zeros):
    mesh = plsc.VectorSubcoreMesh(core_axis_name="c", subcore_axis_name="s")

    @functools.partial(
        pl.kernel,
        out_type=jax.ShapeDtypeStruct((NC, N_PAD, D), jnp.float32),
        mesh=mesh,
        scratch_types=[
            [pltpu.VMEM((K,), jnp.int32) for _ in range(NBUF)],
            [pltpu.VMEM((K,), jnp.int32) for _ in range(NBUF)],
            [pltpu.VMEM((K, D), jnp.float32) for _ in range(NBUF)],
            pltpu.VMEM_SHARED((N_PAD, D), jnp.float32),
            pltpu.VMEM_SHARED((NS * 2 * EPW_MAX,), jnp.int32),
            pltpu.VMEM((EPW_MAX // 4,), jnp.int32),
            [pltpu.SemaphoreType.DMA for _ in range(NBUF)],
            [pltpu.SemaphoreType.DMA for _ in range(NBUF)],
        ],
    )
    def seg_sum(x_hbm, src_hbm, dst_hbm, zeros_hbm, out_hbm,
                src_v, dst_v, rows, acc, sidx, vtmp, sem_g, sem_s):
        cid = lax.axis_index("c")
        sid = lax.axis_index("s")

        zbase = sid * ROWS_PER_TILE
        pltpu.sync_copy(zeros_hbm, acc.at[pl.ds(zbase, ROWS_PER_TILE)])
        epw = jnp.where(cid == 0, EPW0, EPW1)
        nchunk = jnp.where(cid == 0, NCHUNK0, NCHUNK1)
        ebase = jnp.where(cid == 0, sid * EPW0, NS * EPW0 + sid * EPW1)
        tbase = sid * (2 * EPW_MAX)
        H = EPW_MAX // 4
        nh = jnp.where(cid == 0, EPW0 // H + 1, EPW1 // H)

        def stage(h, _):
            n = jnp.minimum(H, epw - h * H)
            pltpu.sync_copy(src_hbm.at[pl.ds(ebase + h * H, n)],
                            vtmp.at[pl.ds(0, n)])
            pltpu.sync_copy(vtmp.at[pl.ds(0, n)],
                            sidx.at[pl.ds(tbase + h * H, n)])
            pltpu.sync_copy(dst_hbm.at[pl.ds(ebase + h * H, n)],
                            vtmp.at[pl.ds(0, n)])
            pltpu.sync_copy(vtmp.at[pl.ds(0, n)],
                            sidx.at[pl.ds(tbase + EPW_MAX + h * H, n)])
            return 0

        lax.fori_loop(0, nh, stage, 0)
        plsc.subcore_barrier()

        def gather(c, b):
            pltpu.sync_copy(sidx.at[pl.ds(tbase + c * K, K)], src_v[b])
            pltpu.async_copy(x_hbm.at[src_v[b]], rows[b], sem_g[b])
            pltpu.sync_copy(sidx.at[pl.ds(tbase + EPW_MAX + c * K, K)],
                            dst_v[b])

        def wait_gather(c, b):
            pltpu.make_async_copy(x_hbm.at[src_v[b]], rows[b],
                                  sem_g[b]).wait()

        def scatter(c, b):
            pltpu.async_copy(rows[b], acc.at[dst_v[b]], sem_s[b], add=True)

        def wait_scatter(c, b):
            pltpu.make_async_copy(rows[b], acc.at[dst_v[b]], sem_s[b]).wait()

        for b in range(NBUF):
            gather(b, b)

        def round_body(r, _):
            base = r * NBUF
            for b in range(NBUF):
                wait_gather(base + b, b)
                scatter(base + b, b)
            for b in range(NBUF):
                wait_scatter(base + b, b)
                gather(base + b + NBUF, b)
            return 0

        nround = nchunk // NBUF
        lax.fori_loop(0, nround - 1, round_body, 0)

        base = (nround - 1) * NBUF
        for b in range(NBUF):
            wait_gather(base + b, b)
            scatter(base + b, b)
        for b in range(NBUF):
            wait_scatter(base + b, b)
        plsc.subcore_barrier()

        pltpu.sync_copy(acc.at[pl.ds(zbase, ROWS_PER_TILE)],
                        out_hbm.at[cid, pl.ds(zbase, ROWS_PER_TILE)])

    return seg_sum(x, src1, dst1, zeros)


BLK = 1000


def _tc_combine(p0, p1, x, wr_t, wt_t, b):
    def body(p0_ref, p1_ref, x_ref, wr_ref, wt_ref, b_ref, o_ref):
        agg = p0_ref[...] + p1_ref[...]
        o_ref[...] = (
            jnp.dot(agg, wr_ref[...], preferred_element_type=jnp.float32)
            + jnp.dot(x_ref[...], wt_ref[...], preferred_element_type=jnp.float32)
            + b_ref[...]
        )

    return pl.pallas_call(
        body,
        grid=(N_NODES // BLK,),
        in_specs=[
            pl.BlockSpec((BLK, D), lambda i: (i, 0)),
            pl.BlockSpec((BLK, D), lambda i: (i, 0)),
            pl.BlockSpec((BLK, D), lambda i: (i, 0)),
            pl.BlockSpec((D, D), lambda i: (0, 0)),
            pl.BlockSpec((D, D), lambda i: (0, 0)),
            pl.BlockSpec((1, D), lambda i: (0, 0)),
        ],
        out_specs=pl.BlockSpec((BLK, D), lambda i: (i, 0)),
        out_shape=jax.ShapeDtypeStruct((N_NODES, D), jnp.float32),
    )(p0, p1, x, wr_t, wt_t, b)


def kernel(x, edge_index, W_rel, b_rel, W_root):
    src = edge_index[0].astype(jnp.int32)
    dst = edge_index[1].astype(jnp.int32)
    npad = E_PAD - N_EDGES
    src1 = jnp.concatenate([src, jnp.zeros((npad,), jnp.int32)])
    dst1 = jnp.concatenate([dst, jnp.full((npad,), N_NODES, jnp.int32)])
    zeros = jnp.zeros((ROWS_PER_TILE, D), jnp.float32)
    partials = _sc_segment_sum(x, src1, dst1, zeros)
    return _tc_combine(partials[0, :N_NODES], partials[1, :N_NODES], x,
                       W_rel.T, W_root.T, b_rel.reshape(1, D))


# speedup vs baseline: 1.1198x; 1.1198x over previous
import functools

import jax
import jax.numpy as jnp
from jax import lax
from jax.experimental import pallas as pl
from jax.experimental.pallas import tpu as pltpu
from jax.experimental.pallas import tpu_sc as plsc

N_NODES = 10000
N_EDGES = 320000
D = 128

NC = 2
NS = 16
NW = NC * NS
K = 80
NBUF = 2
NCHUNK0 = 152
NCHUNK1 = 100
EPW0 = NCHUNK0 * K
EPW1 = NCHUNK1 * K
EPW_MAX = EPW0
E_PAD = NS * (EPW0 + EPW1)
N_PAD = 10240
ROWS_PER_TILE = N_PAD // NS


def _sc_segment_sum(x, src1, dst1, zeros):
    mesh = plsc.VectorSubcoreMesh(core_axis_name="c", subcore_axis_name="s")

    @functools.partial(
        pl.kernel,
        out_type=jax.ShapeDtypeStruct((NC, N_PAD, D), jnp.float32),
        mesh=mesh,
        scratch_types=[
            [pltpu.VMEM((K,), jnp.int32) for _ in range(NBUF)],
            [pltpu.VMEM((K,), jnp.int32) for _ in range(NBUF)],
            [pltpu.VMEM((K, D), jnp.float32) for _ in range(NBUF)],
            pltpu.VMEM_SHARED((N_PAD, D), jnp.float32),
            pltpu.VMEM_SHARED((NS * 2 * EPW_MAX,), jnp.int32),
            pltpu.VMEM((EPW_MAX // 4,), jnp.int32),
            [pltpu.SemaphoreType.DMA for _ in range(NBUF)],
            [pltpu.SemaphoreType.DMA for _ in range(NBUF)],
        ],
    )
    def seg_sum(x_hbm, src_hbm, dst_hbm, zeros_hbm, out_hbm,
                src_v, dst_v, rows, acc, sidx, vtmp, sem_g, sem_s):
        cid = lax.axis_index("c")
        sid = lax.axis_index("s")

        zbase = sid * ROWS_PER_TILE
        pltpu.sync_copy(zeros_hbm, acc.at[pl.ds(zbase, ROWS_PER_TILE)])
        epw = jnp.where(cid == 0, EPW0, EPW1)
        nchunk = jnp.where(cid == 0, NCHUNK0, NCHUNK1)
        ebase = jnp.where(cid == 0, sid * EPW0, NS * EPW0 + sid * EPW1)
        tbase = sid * (2 * EPW_MAX)
        H = EPW_MAX // 4
        nh = jnp.where(cid == 0, EPW0 // H, EPW1 // H + 1)

        def stage(h, _):
            n = jnp.minimum(H, epw - h * H)
            pltpu.sync_copy(src_hbm.at[pl.ds(ebase + h * H, n)],
                            vtmp.at[pl.ds(0, n)])
            pltpu.sync_copy(vtmp.at[pl.ds(0, n)],
                            sidx.at[pl.ds(tbase + h * H, n)])
            pltpu.sync_copy(dst_hbm.at[pl.ds(ebase + h * H, n)],
                            vtmp.at[pl.ds(0, n)])
            pltpu.sync_copy(vtmp.at[pl.ds(0, n)],
                            sidx.at[pl.ds(tbase + EPW_MAX + h * H, n)])
            return 0

        lax.fori_loop(0, nh, stage, 0)
        plsc.subcore_barrier()

        def gather(c, b):
            pltpu.sync_copy(sidx.at[pl.ds(tbase + c * K, K)], src_v[b])
            pltpu.async_copy(x_hbm.at[src_v[b]], rows[b], sem_g[b])
            pltpu.sync_copy(sidx.at[pl.ds(tbase + EPW_MAX + c * K, K)],
                            dst_v[b])

        def wait_gather(c, b):
            pltpu.make_async_copy(x_hbm.at[src_v[b]], rows[b],
                                  sem_g[b]).wait()

        def scatter(c, b):
            pltpu.async_copy(rows[b], acc.at[dst_v[b]], sem_s[b], add=True)

        def wait_scatter(c, b):
            pltpu.make_async_copy(rows[b], acc.at[dst_v[b]], sem_s[b]).wait()

        for b in range(NBUF):
            gather(b, b)

        def round_body(r, _):
            base = r * NBUF
            for b in range(NBUF):
                wait_gather(base + b, b)
                scatter(base + b, b)
            for b in range(NBUF):
                wait_scatter(base + b, b)
                gather(base + b + NBUF, b)
            return 0

        nround = nchunk // NBUF
        lax.fori_loop(0, nround - 1, round_body, 0)

        base = (nround - 1) * NBUF
        for b in range(NBUF):
            wait_gather(base + b, b)
            scatter(base + b, b)
        for b in range(NBUF):
            wait_scatter(base + b, b)
        plsc.subcore_barrier()

        pltpu.sync_copy(acc.at[pl.ds(zbase, ROWS_PER_TILE)],
                        out_hbm.at[cid, pl.ds(zbase, ROWS_PER_TILE)])

    return seg_sum(x, src1, dst1, zeros)


BLK = 1000


def _tc_combine(p0, p1, x, wr_t, wt_t, b):
    def body(p0_ref, p1_ref, x_ref, wr_ref, wt_ref, b_ref, o_ref):
        agg = p0_ref[...] + p1_ref[...]
        o_ref[...] = (
            jnp.dot(agg, wr_ref[...], preferred_element_type=jnp.float32)
            + jnp.dot(x_ref[...], wt_ref[...], preferred_element_type=jnp.float32)
            + b_ref[...]
        )

    return pl.pallas_call(
        body,
        grid=(N_NODES // BLK,),
        in_specs=[
            pl.BlockSpec((BLK, D), lambda i: (i, 0)),
            pl.BlockSpec((BLK, D), lambda i: (i, 0)),
            pl.BlockSpec((BLK, D), lambda i: (i, 0)),
            pl.BlockSpec((D, D), lambda i: (0, 0)),
            pl.BlockSpec((D, D), lambda i: (0, 0)),
            pl.BlockSpec((1, D), lambda i: (0, 0)),
        ],
        out_specs=pl.BlockSpec((BLK, D), lambda i: (i, 0)),
        out_shape=jax.ShapeDtypeStruct((N_NODES, D), jnp.float32),
    )(p0, p1, x, wr_t, wt_t, b)


def kernel(x, edge_index, W_rel, b_rel, W_root):
    src = edge_index[0].astype(jnp.int32)
    dst = edge_index[1].astype(jnp.int32)
    npad = E_PAD - N_EDGES
    src1 = jnp.concatenate([src, jnp.zeros((npad,), jnp.int32)])
    dst1 = jnp.concatenate([dst, jnp.full((npad,), N_NODES, jnp.int32)])
    zeros = jnp.zeros((ROWS_PER_TILE, D), jnp.float32)
    partials = _sc_segment_sum(x, src1, dst1, zeros)
    return _tc_combine(partials[0, :N_NODES], partials[1, :N_NODES], x,
                       W_rel.T, W_root.T, b_rel.reshape(1, D))
